# baseline (speedup 1.0000x reference)
import functools
import jax
import jax.numpy as jnp
from jax import lax
from jax.experimental import pallas as pl
from jax.experimental.pallas import tpu as pltpu
from jax.experimental.pallas import tpu_sc as plsc

_B, _C, _H, _W = 32, 768, 32, 32
_HW = _H * _W
_P = 32
_DIM = 384
_PAD = 33


def _sc_body(feats_ref, pe_x_ref, pe_y_ref, glb_ref, out_ref,
             slab, px_buf, out_buf, row_buf, glb_buf):
    cid = lax.axis_index("c")
    sid = lax.axis_index("s")
    wid = sid * 2 + cid
    t0 = wid * _P

    pltpu.sync_copy(pe_x_ref, px_buf)
    pltpu.sync_copy(pe_y_ref.at[wid], row_buf)
    pltpu.sync_copy(glb_ref, glb_buf)
    pltpu.sync_copy(glb_buf, out_ref.at[wid, pl.ds(0, 1)])

    rowv = [row_buf[pl.ds(16 * i, 16)] for i in range(_DIM // 16)]
    iota = lax.iota(jnp.int32, 16)
    rows = [iota + 16 * j for j in range(_C // 16)]

    def per_b(b, carry):
        pltpu.sync_copy(feats_ref.at[b, :, pl.ds(t0, _P)],
                        slab.at[:, pl.ds(0, _P)])

        def per_t(t, inner):
            ct = jnp.full((16,), t, jnp.int32)
            for j in range(_DIM // 16):
                v = plsc.load_gather(slab, [rows[j], ct])
                p = px_buf[t, pl.ds(16 * j, 16)]
                out_buf[t, pl.ds(16 * j, 16)] = v + p
            for j in range(_DIM // 16, _C // 16):
                v = plsc.load_gather(slab, [rows[j], ct])
                out_buf[t, pl.ds(16 * j, 16)] = v + rowv[j - _DIM // 16]
            return inner
        lax.fori_loop(0, _P, per_t, 0)

        pltpu.sync_copy(out_buf, out_ref.at[b, pl.ds(1 + t0, _P)])
        return carry
    lax.fori_loop(0, _B, per_b, 0)


def kernel(feats, pe_x_table, pe_y_table, glb_table):
    b, c, h, w = feats.shape
    hw = h * w
    feats2 = feats.reshape(b, c, hw)

    mesh = plsc.VectorSubcoreMesh(core_axis_name="c", subcore_axis_name="s")
    k = functools.partial(
        pl.kernel,
        mesh=mesh,
        compiler_params=pltpu.CompilerParams(
            use_tc_tiling_on_sc=False, needs_layout_passes=False),
        out_type=jax.ShapeDtypeStruct((b, 1 + hw, c), feats.dtype),
        scratch_types=[
            pltpu.VMEM((_C, _PAD), jnp.float32),
            pltpu.VMEM((_W, _DIM), jnp.float32),
            pltpu.VMEM((_P, _C), jnp.float32),
            pltpu.VMEM((_DIM,), jnp.float32),
            pltpu.VMEM((1, _C), jnp.float32),
        ],
    )(_sc_body)
    return k(feats2, pe_x_table, pe_y_table, glb_table)


# speedup vs baseline: 1.5195x; 1.5195x over previous
import functools
import jax
import jax.numpy as jnp
from jax import lax
from jax.experimental import pallas as pl
from jax.experimental.pallas import tpu as pltpu
from jax.experimental.pallas import tpu_sc as plsc

_B, _C, _H, _W = 32, 768, 32, 32
_HW = _H * _W
_P = 32
_DIM = 384
_PAD = 40
_NJ = _C // 16


def _sc_body(feats_ref, pe_x_ref, pe_y_ref, glb_ref, out_ref,
             slab_a, slab_b, out_a, out_b, px_buf, row_buf, glb_buf,
             sem_ia, sem_ib, sem_oa, sem_ob):
    cid = lax.axis_index("c")
    sid = lax.axis_index("s")
    wid = sid * 2 + cid
    t0 = wid * _P

    pltpu.sync_copy(pe_x_ref, px_buf)
    pltpu.sync_copy(pe_y_ref.at[wid], row_buf)
    pltpu.sync_copy(glb_ref, glb_buf)
    pltpu.sync_copy(glb_buf, out_ref.at[wid, pl.ds(0, 1)])

    rowv = [row_buf[pl.ds(16 * i, 16)] for i in range(_DIM // 16)]
    iota = lax.iota(jnp.int32, 16)
    v_lane = iota * _PAD
    zeros = jnp.zeros((16,), jnp.int32)

    def in_src(b):
        return feats_ref.at[b, :, pl.ds(t0, _P)]

    def out_dst(b):
        return out_ref.at[b, pl.ds(1 + t0, _P)]

    def compute(slab, out_buf):
        @plsc.parallel_loop(0, _P, unroll=2)
        def per_t(t):
            vt = v_lane + t
            for j in range(_NJ):
                idx = vt + (16 * _PAD) * j
                v = plsc.load_gather(slab, [zeros, idx])
                if j < _DIM // 16:
                    p = px_buf[t, pl.ds(16 * j, 16)]
                else:
                    p = rowv[j - _DIM // 16]
                out_buf[t, pl.ds(16 * j, 16)] = v + p

    pltpu.async_copy(in_src(0), slab_a.at[:, pl.ds(0, _P)], sem_ia)

    def pair(i, carry):
        b_a = 2 * i
        b_b = b_a + 1

        @pl.when(i > 0)
        def _():
            pltpu.make_async_copy(out_a, out_dst(0), sem_oa).wait()
        pltpu.make_async_copy(in_src(0), slab_a.at[:, pl.ds(0, _P)],
                              sem_ia).wait()
        pltpu.async_copy(in_src(b_b), slab_b.at[:, pl.ds(0, _P)], sem_ib)
        compute(slab_a, out_a)
        pltpu.async_copy(out_a, out_dst(b_a), sem_oa)

        @pl.when(i > 0)
        def _():
            pltpu.make_async_copy(out_b, out_dst(0), sem_ob).wait()
        pltpu.make_async_copy(in_src(0), slab_b.at[:, pl.ds(0, _P)],
                              sem_ib).wait()

        @pl.when(i < _B // 2 - 1)
        def _():
            pltpu.async_copy(in_src(b_b + 1), slab_a.at[:, pl.ds(0, _P)],
                             sem_ia)
        compute(slab_b, out_b)
        pltpu.async_copy(out_b, out_dst(b_b), sem_ob)
        return carry

    lax.fori_loop(0, _B // 2, pair, 0)
    pltpu.make_async_copy(out_a, out_dst(0), sem_oa).wait()
    pltpu.make_async_copy(out_b, out_dst(0), sem_ob).wait()


def kernel(feats, pe_x_table, pe_y_table, glb_table):
    b, c, h, w = feats.shape
    hw = h * w
    feats2 = feats.reshape(b, c, hw)

    mesh = plsc.VectorSubcoreMesh(core_axis_name="c", subcore_axis_name="s")
    k = functools.partial(
        pl.kernel,
        mesh=mesh,
        compiler_params=pltpu.CompilerParams(
            use_tc_tiling_on_sc=False, needs_layout_passes=False),
        out_type=jax.ShapeDtypeStruct((b, 1 + hw, c), feats.dtype),
        scratch_types=[
            pltpu.VMEM((_C, _PAD), jnp.float32),
            pltpu.VMEM((_C, _PAD), jnp.float32),
            pltpu.VMEM((_P, _C), jnp.float32),
            pltpu.VMEM((_P, _C), jnp.float32),
            pltpu.VMEM((_W, _DIM), jnp.float32),
            pltpu.VMEM((_DIM,), jnp.float32),
            pltpu.VMEM((1, _C), jnp.float32),
            pltpu.SemaphoreType.DMA,
            pltpu.SemaphoreType.DMA,
            pltpu.SemaphoreType.DMA,
            pltpu.SemaphoreType.DMA,
        ],
    )(_sc_body)
    return k(feats2, pe_x_table, pe_y_table, glb_table)
